# baseline (speedup 1.0000x reference)
import jax
import jax.numpy as jnp
from jax import lax
from jax.experimental import pallas as pl
from jax.experimental.pallas import tpu as pltpu
from jax.experimental.pallas import tpu_sc as plsc

B, L, D = 4096, 200, 64
V = 1000000
NC, NS = 2, 16
NW = NC * NS
TOK = B * L
RPW = B // NW
POSN = L * D
SK = 129
NBLK = V // 128
BPW = NBLK // NW
LGS = tuple(range(0, 192, 16)) + (184,)


def _k0_body(tabT_hbm, tail_hbm, emb2_hbm, sk0, sk1, to0, to1, i0, i1, o0, o1):
    skb = (sk0, sk1)
    tob = (to0, to1)
    isem = (i0, i1)
    osem = (o0, o1)
    wid = lax.axis_index("s") * NC + lax.axis_index("c")
    ii = lax.iota(jnp.int32, 16)

    def blk_in(c, s):
        off = pl.multiple_of(c * 128, 128)
        pltpu.async_copy(tabT_hbm.at[pl.ds(0, 64), pl.ds(off, 128)],
                         skb[s].at[pl.ds(0, 64), pl.ds(0, 128)], isem[s])

    def blk_in_wait(c, s):
        off = pl.multiple_of(c * 128, 128)
        pltpu.make_async_copy(tabT_hbm.at[pl.ds(0, 64), pl.ds(off, 128)],
                              skb[s].at[pl.ds(0, 64), pl.ds(0, 128)],
                              isem[s]).wait()

    def blk_out(c, s):
        off = pl.multiple_of(c * 64, 64)
        pltpu.async_copy(tob[s], emb2_hbm.at[pl.ds(off, 64)], osem[s])

    def blk_out_wait(c, s):
        off = pl.multiple_of(c * 64, 64)
        pltpu.make_async_copy(tob[s], emb2_hbm.at[pl.ds(off, 64)],
                              osem[s]).wait()

    def repack(s):
        flat = skb[s]

        @pl.loop(0, 64, unroll=4)
        def _j(j):
            v0 = 2 * j
            for h in range(4):
                tob[s][j, pl.ds(16 * h, 16)] = plsc.load_gather(
                    flat, [ii + 16 * h, jnp.broadcast_to(v0, (16,))])
                tob[s][j, pl.ds(64 + 16 * h, 16)] = plsc.load_gather(
                    flat, [ii + 16 * h, jnp.broadcast_to(v0 + 1, (16,))])

    base = wid * BPW

    def gstep(i, s, o):
        c = base + i

        @pl.when(i + 1 < BPW)
        def _nxt():
            @pl.when(i >= 1)
            def _w():
                blk_out_wait(c - 1, o)
            blk_in(c + 1, o)
        blk_in_wait(c, s)
        repack(s)
        blk_out(c, s)

    blk_in(base, 0)

    @pl.loop(0, BPW, step=2)
    def _c(i):
        gstep(i, 0, 1)
        gstep(i + 1, 1, 0)

    blk_out_wait(base + BPW - 2, (BPW - 2) % 2)
    blk_out_wait(base + BPW - 1, (BPW - 1) % 2)

    @pl.when(wid < 4)
    def _tail_full():
        c = NW * BPW + wid
        blk_in(c, 0)
        blk_in_wait(c, 0)
        repack(0)
        pltpu.sync_copy(tob[0], emb2_hbm.at[pl.ds(c * 64, 64)])

    @pl.when(wid == 4)
    def _tail_half():
        pltpu.sync_copy(tail_hbm, to0.at[pl.ds(0, 32)])
        pltpu.sync_copy(to0.at[pl.ds(0, 32)], emb2_hbm.at[pl.ds(499968, 32)])


def _k1_body(emb_hbm, x_hbm, pos_hbm, out_hbm,
             posL,
             idx0, idx1, q0, q1, pb0, pb1, r0, r1, t0, t1,
             i0, i1, g0, g1, o0, o1):
    idx = (idx0, idx1)
    q = (q0, q1)
    pb = (pb0, pb1)
    rows = (r0, r1)
    tout = (t0, t1)
    isem = (i0, i1)
    gsem = (g0, g1)
    osem = (o0, o1)

    wid = lax.axis_index("s") * NC + lax.axis_index("c")
    row_w = wid * RPW

    def idx_dma(c, s):
        pltpu.async_copy(x_hbm.at[pl.ds((row_w + c) * L, L)], idx[s], isem[s])

    def idx_wait(c, s):
        pltpu.make_async_copy(x_hbm.at[pl.ds((row_w + c) * L, L)], idx[s],
                              isem[s]).wait()

    def prep(s):
        for lg in LGS:
            v = idx[s][pl.ds(lg, 16)]
            q[s][pl.ds(lg, 16)] = lax.shift_right_logical(v, 1)
            pb[s][pl.ds(lg, 16)] = (v & 1) * 64

    def gather(c, s):
        pltpu.async_copy(emb_hbm.at[q[s]], rows[s], gsem[s])

    def gather_wait(c, s):
        pltpu.make_async_copy(emb_hbm.at[q[s]], rows[s], gsem[s]).wait()

    def out_dma(c, s):
        pltpu.async_copy(tout[s], out_hbm.at[row_w + c], osem[s])

    def out_wait(c, s):
        pltpu.make_async_copy(tout[s], out_hbm.at[row_w + c],
                              osem[s]).wait()

    def select_add(s):
        @pl.loop(0, L, unroll=2)
        def _l(t):
            o = pb[s][pl.ds(t, 16)][0]
            pbase = t * D
            for c4 in range(4):
                tout[s][t, pl.ds(c4 * 16, 16)] = (
                    rows[s][t, pl.ds(o + c4 * 16, 16)]
                    + posL[pl.ds(pbase + c4 * 16, 16)]
                )

    pltpu.sync_copy(pos_hbm.at[pl.ds(0, POSN)], posL)

    def _step(c, s, o):
        @pl.when(c + 1 < RPW)
        def _launch():
            idx_wait(c + 1, o)
            prep(o)
            gather(c + 1, o)

        gather_wait(c, s)

        @pl.when(c >= 2)
        def _drain():
            out_wait(c - 2, s)

        select_add(s)
        out_dma(c, s)

        @pl.when(c + 2 < RPW)
        def _next_idx():
            idx_dma(c + 2, s)

    idx_dma(0, 0)
    idx_wait(0, 0)
    prep(0)
    gather(0, 0)
    idx_dma(1, 1)

    @pl.loop(0, RPW, step=2)
    def _chunk(c):
        _step(c, 0, 1)
        _step(c + 1, 1, 0)

    out_wait(RPW - 2, (RPW - 2) % 2)
    out_wait(RPW - 1, (RPW - 1) % 2)


@jax.jit
def kernel(x, embed_table, pos_table):
    tabT = embed_table.T
    x_flat = x.reshape(TOK).astype(jnp.int32)
    mesh = plsc.VectorSubcoreMesh(core_axis_name="c", subcore_axis_name="s",
                                  num_cores=NC, num_subcores=NS)
    cp = pltpu.CompilerParams(use_tc_tiling_on_sc=True,
                              needs_layout_passes=False)

    emb2 = pl.kernel(
        _k0_body,
        out_type=jax.ShapeDtypeStruct((V // 2, 128), jnp.float32),
        mesh=mesh,
        compiler_params=cp,
        scratch_types=[
            pltpu.VMEM((64, SK), jnp.float32),
            pltpu.VMEM((64, SK), jnp.float32),
            pltpu.VMEM((64, 128), jnp.float32),
            pltpu.VMEM((64, 128), jnp.float32),
            pltpu.SemaphoreType.DMA,
            pltpu.SemaphoreType.DMA,
            pltpu.SemaphoreType.DMA,
            pltpu.SemaphoreType.DMA,
        ],
    )(tabT, jax.lax.slice(embed_table, (999936, 0), (V, D)).reshape(32, 128))

    out = pl.kernel(
        _k1_body,
        out_type=jax.ShapeDtypeStruct((B, L, D), jnp.float32),
        mesh=mesh,
        compiler_params=cp,
        scratch_types=[
            pltpu.VMEM((POSN,), jnp.float32),
            pltpu.VMEM((L,), jnp.int32),
            pltpu.VMEM((L,), jnp.int32),
            pltpu.VMEM((L,), jnp.int32),
            pltpu.VMEM((L,), jnp.int32),
            pltpu.VMEM((L + 16,), jnp.int32),
            pltpu.VMEM((L + 16,), jnp.int32),
            pltpu.VMEM((L, 128), jnp.float32),
            pltpu.VMEM((L, 128), jnp.float32),
            pltpu.VMEM((L, D), jnp.float32),
            pltpu.VMEM((L, D), jnp.float32),
            pltpu.SemaphoreType.DMA,
            pltpu.SemaphoreType.DMA,
            pltpu.SemaphoreType.DMA,
            pltpu.SemaphoreType.DMA,
            pltpu.SemaphoreType.DMA,
            pltpu.SemaphoreType.DMA,
        ],
    )(emb2, x_flat, pos_table.reshape(-1))
    return out


# speedup vs baseline: 1.7302x; 1.7302x over previous
import jax
import jax.numpy as jnp
from jax import lax
from jax.experimental import pallas as pl
from jax.experimental.pallas import tpu as pltpu
from jax.experimental.pallas import tpu_sc as plsc

B, L, D = 4096, 200, 64
NC, NS = 2, 16
NW = NC * NS
TOK = B * L
RPW = B // NW
VPD = D // 16


def _body(emb_hbm, x_hbm, pos_hbm, out_hbm,
          pos_v, idx0, idx1, r0, r1, i0, i1, g0, g1, o0, o1):
    idx = (idx0, idx1)
    rows = (r0, r1)
    isem = (i0, i1)
    gsem = (g0, g1)
    osem = (o0, o1)

    wid = lax.axis_index("s") * NC + lax.axis_index("c")
    row_w = wid * RPW

    def idx_dma(c, s):
        pltpu.async_copy(x_hbm.at[pl.ds((row_w + c) * L, L)], idx[s], isem[s])

    def idx_wait(c, s):
        pltpu.make_async_copy(x_hbm.at[pl.ds((row_w + c) * L, L)], idx[s],
                              isem[s]).wait()

    def gather(c, s):
        pltpu.async_copy(emb_hbm.at[idx[s]], rows[s], gsem[s])

    def gather_wait(c, s):
        pltpu.make_async_copy(emb_hbm.at[idx[s]], rows[s], gsem[s]).wait()

    def out_dma(c, s):
        pltpu.async_copy(rows[s], out_hbm.at[row_w + c], osem[s])

    def out_wait(c, s):
        pltpu.make_async_copy(rows[s], out_hbm.at[row_w + c],
                              osem[s]).wait()

    def add_pos(s):
        @pl.loop(0, L, unroll=4)
        def _l(j):
            for v in range(VPD):
                rows[s][j, pl.ds(v * 16, 16)] = (
                    rows[s][j, pl.ds(v * 16, 16)]
                    + pos_v[j, pl.ds(v * 16, 16)]
                )

    pltpu.sync_copy(pos_hbm.at[pl.ds(0, L)], pos_v)

    def _step(c, s, o):
        @pl.when(c + 1 < RPW)
        def _launch():
            idx_wait(c + 1, o)

            @pl.when(c >= 1)
            def _drain():
                out_wait(c - 1, o)
            gather(c + 1, o)

        gather_wait(c, s)
        add_pos(s)
        out_dma(c, s)

        @pl.when(c + 2 < RPW)
        def _next_idx():
            idx_dma(c + 2, s)

    idx_dma(0, 0)
    idx_wait(0, 0)
    gather(0, 0)
    idx_dma(1, 1)

    @pl.loop(0, RPW, step=2)
    def _chunk(c):
        _step(c, 0, 1)
        _step(c + 1, 1, 0)

    out_wait(RPW - 2, 0)
    out_wait(RPW - 1, 1)


@jax.jit
def kernel(x, embed_table, pos_table):
    x_flat = x.reshape(TOK).astype(jnp.int32)
    mesh = plsc.VectorSubcoreMesh(core_axis_name="c", subcore_axis_name="s",
                                  num_cores=NC, num_subcores=NS)
    out = pl.kernel(
        _body,
        out_type=jax.ShapeDtypeStruct((B, L, D), jnp.float32),
        mesh=mesh,
        compiler_params=pltpu.CompilerParams(use_tc_tiling_on_sc=False),
        scratch_types=[
            pltpu.VMEM((L, D), jnp.float32),
            pltpu.VMEM((L,), jnp.int32),
            pltpu.VMEM((L,), jnp.int32),
            pltpu.VMEM((L, D), jnp.float32),
            pltpu.VMEM((L, D), jnp.float32),
            pltpu.SemaphoreType.DMA,
            pltpu.SemaphoreType.DMA,
            pltpu.SemaphoreType.DMA,
            pltpu.SemaphoreType.DMA,
            pltpu.SemaphoreType.DMA,
            pltpu.SemaphoreType.DMA,
        ],
    )(embed_table, x_flat, pos_table)
    return out


# speedup vs baseline: 2.0214x; 1.1683x over previous
import functools

import jax
import jax.numpy as jnp
from jax import lax
from jax.experimental import pallas as pl
from jax.experimental.pallas import tpu as pltpu
from jax.experimental.pallas import tpu_sc as plsc

B, L, D = 4096, 200, 64
NC, NS = 2, 16
NW = NC * NS
TOK = B * L
TPW = TOK // NW
T = 400
NCH = TPW // T
VPD = D // 16


def _body(emb_hbm, x_hbm, pos_hbm, out_hbm, idx_v, rows_v, pos_v, gsem):
    wid = lax.axis_index("s") * NC + lax.axis_index("c")
    base_w = wid * TPW
    pltpu.sync_copy(pos_hbm.at[pl.ds(0, L)], pos_v)

    @pl.loop(0, NCH)
    def _chunk(i):
        base = base_w + i * T
        pltpu.sync_copy(x_hbm.at[pl.ds(base, T)], idx_v)
        pltpu.async_copy(emb_hbm.at[idx_v], rows_v, gsem).wait()

        @pl.loop(0, L)
        def _add(j):
            for c in range(VPD):
                p = pos_v[j, pl.ds(c * 16, 16)]
                for r in range(T // L):
                    t = r * L + j
                    rows_v[t, pl.ds(c * 16, 16)] = (
                        rows_v[t, pl.ds(c * 16, 16)] + p
                    )

        pltpu.sync_copy(rows_v, out_hbm.at[pl.ds(base, T)])


@jax.jit
def kernel(x, embed_table, pos_table):
    x_flat = x.reshape(TOK).astype(jnp.int32)
    mesh = plsc.VectorSubcoreMesh(core_axis_name="c", subcore_axis_name="s",
                                  num_cores=NC, num_subcores=NS)
    out = pl.kernel(
        _body,
        out_type=jax.ShapeDtypeStruct((TOK, D), jnp.float32),
        mesh=mesh,
        compiler_params=pltpu.CompilerParams(use_tc_tiling_on_sc=False),
        scratch_types=[
            pltpu.VMEM((T,), jnp.int32),
            pltpu.VMEM((T, D), jnp.float32),
            pltpu.VMEM((L, D), jnp.float32),
            pltpu.SemaphoreType.DMA,
        ],
    )(embed_table, x_flat, pos_table)
    return out.reshape(B, L, D)
